# baseline (speedup 1.0000x reference)
import jax
import jax.numpy as jnp
from jax.experimental import pallas as pl
from jax.experimental.pallas import tpu as pltpu


def _fused_mlp_kernel(x_ref, w1_ref, b1_ref, w2_ref, b2_ref, o_ref):
    h = jax.lax.dot_general(
        x_ref[...], w1_ref[...],
        dimension_numbers=(((1,), (0,)), ((), ())),
        preferred_element_type=jnp.float32,
    )
    h = jnp.maximum(h + b1_ref[...], 0.0)
    o = jax.lax.dot_general(
        h, w2_ref[...],
        dimension_numbers=(((1,), (0,)), ((), ())),
        preferred_element_type=jnp.float32,
    )
    o_ref[...] = o + b2_ref[...]


def kernel(x_transaction, x_merchant, x_category, edge_index_belongs_to, edge_index_has_category, W_enc_t, b_enc_t, W_enc_m, b_enc_m, W_enc_c, b_enc_c, lin_l_bm_0, bias_bm_0, lin_r_bm_0, lin_l_tc_0, bias_tc_0, lin_r_tc_0, lin_l_bm_1, bias_bm_1, lin_r_bm_1, lin_l_tc_1, bias_tc_1, lin_r_tc_1, W_cls, b_cls):
    NT, D = x_transaction.shape
    H = W_enc_t.shape[1]
    OUT = W_cls.shape[1]
    OUTP = ((OUT + 511) // 512) * 512

    BR = 2000
    grid = (pl.cdiv(NT, BR),)

    b1 = b_enc_t.reshape(1, H)
    W2 = jnp.pad(W_cls, ((0, 0), (0, OUTP - OUT)))
    b2 = jnp.pad(b_cls, (0, OUTP - OUT)).reshape(1, OUTP)

    out = pl.pallas_call(
        _fused_mlp_kernel,
        grid=grid,
        in_specs=[
            pl.BlockSpec((BR, D), lambda i: (i, 0)),
            pl.BlockSpec((D, H), lambda i: (0, 0)),
            pl.BlockSpec((1, H), lambda i: (0, 0)),
            pl.BlockSpec((H, OUTP), lambda i: (0, 0)),
            pl.BlockSpec((1, OUTP), lambda i: (0, 0)),
        ],
        out_specs=pl.BlockSpec((BR, OUTP), lambda i: (i, 0)),
        out_shape=jax.ShapeDtypeStruct((NT, OUTP), jnp.float32),
        compiler_params=pltpu.CompilerParams(
            dimension_semantics=("parallel",),
        ),
    )(x_transaction, W_enc_t, b1, W2, b2)
    return out[:, :OUT]


# speedup vs baseline: 3.5145x; 3.5145x over previous
import jax
import jax.numpy as jnp
from jax.experimental import pallas as pl
from jax.experimental.pallas import tpu as pltpu


def _fused_mlp_kernel(x_ref, w1_ref, b1_ref, w2_ref, b2_ref, o_ref):
    h = jax.lax.dot_general(
        x_ref[...], w1_ref[...],
        dimension_numbers=(((1,), (0,)), ((), ())),
        preferred_element_type=jnp.float32,
    )
    h = jnp.maximum(h + b1_ref[...], 0.0)
    o = jax.lax.dot_general(
        h, w2_ref[...],
        dimension_numbers=(((1,), (0,)), ((), ())),
        preferred_element_type=jnp.float32,
    )
    o_ref[...] = o + b2_ref[...]


def kernel(x_transaction, x_merchant, x_category, edge_index_belongs_to, edge_index_has_category, W_enc_t, b_enc_t, W_enc_m, b_enc_m, W_enc_c, b_enc_c, lin_l_bm_0, bias_bm_0, lin_r_bm_0, lin_l_tc_0, bias_tc_0, lin_r_tc_0, lin_l_bm_1, bias_bm_1, lin_r_bm_1, lin_l_tc_1, bias_tc_1, lin_r_tc_1, W_cls, b_cls):
    NT, D = x_transaction.shape
    H = W_enc_t.shape[1]
    OUT = W_cls.shape[1]
    OUTP = ((OUT + 511) // 512) * 512

    BR = 2000
    grid = (pl.cdiv(NT, BR),)

    b1 = b_enc_t.reshape(1, H)
    W2 = jnp.pad(W_cls, ((0, 0), (0, OUTP - OUT)))
    b2 = jnp.pad(b_cls, (0, OUTP - OUT)).reshape(1, OUTP)

    out = pl.pallas_call(
        _fused_mlp_kernel,
        grid=grid,
        in_specs=[
            pl.BlockSpec((BR, D), lambda i: (i, 0)),
            pl.BlockSpec((D, H), lambda i: (0, 0)),
            pl.BlockSpec((1, H), lambda i: (0, 0)),
            pl.BlockSpec((H, OUTP), lambda i: (0, 0)),
            pl.BlockSpec((1, OUTP), lambda i: (0, 0)),
        ],
        out_specs=pl.BlockSpec((BR, OUTP), lambda i: (i, 0)),
        out_shape=jax.ShapeDtypeStruct((NT, OUT), jnp.float32),
        compiler_params=pltpu.CompilerParams(
            dimension_semantics=("parallel",),
        ),
    )(x_transaction, W_enc_t, b1, W2, b2)
    return out


# speedup vs baseline: 3.6424x; 1.0364x over previous
import jax
import jax.numpy as jnp
from jax.experimental import pallas as pl
from jax.experimental.pallas import tpu as pltpu

_BR = 2000
_Q = 4


def _fused_mlp_kernel(x_ref, w1_ref, b1_ref, w2_ref, b2_ref, o_hbm, scratch, sems):
    i = pl.program_id(0)
    n = pl.num_programs(0)
    slot = jax.lax.rem(i, _Q)

    @pl.when(i >= _Q)
    def _wait_prev():
        j = i - _Q
        pltpu.make_async_copy(
            scratch.at[slot], o_hbm.at[pl.ds(j * _BR, _BR), :], sems.at[slot]
        ).wait()

    h = jax.lax.dot_general(
        x_ref[...], w1_ref[...],
        dimension_numbers=(((1,), (0,)), ((), ())),
        preferred_element_type=jnp.float32,
    )
    h = jnp.maximum(h + b1_ref[...], 0.0)
    o = jax.lax.dot_general(
        h, w2_ref[...],
        dimension_numbers=(((1,), (0,)), ((), ())),
        preferred_element_type=jnp.float32,
    )
    scratch[slot] = o + b2_ref[...]

    pltpu.make_async_copy(
        scratch.at[slot], o_hbm.at[pl.ds(i * _BR, _BR), :], sems.at[slot]
    ).start()

    @pl.when(i == n - 1)
    def _drain():
        for q in range(1, _Q + 1):
            j = n - 1 - _Q + q
            s = jax.lax.rem(jnp.int32(j), _Q)
            pltpu.make_async_copy(
                scratch.at[s], o_hbm.at[pl.ds(j * _BR, _BR), :], sems.at[s]
            ).wait()


def kernel(x_transaction, x_merchant, x_category, edge_index_belongs_to, edge_index_has_category, W_enc_t, b_enc_t, W_enc_m, b_enc_m, W_enc_c, b_enc_c, lin_l_bm_0, bias_bm_0, lin_r_bm_0, lin_l_tc_0, bias_tc_0, lin_r_tc_0, lin_l_bm_1, bias_bm_1, lin_r_bm_1, lin_l_tc_1, bias_tc_1, lin_r_tc_1, W_cls, b_cls):
    NT, D = x_transaction.shape
    H = W_enc_t.shape[1]
    OUT = W_cls.shape[1]

    grid = (NT // _BR,)

    b1 = b_enc_t.reshape(1, H)
    b2 = b_cls.reshape(1, OUT)

    return pl.pallas_call(
        _fused_mlp_kernel,
        grid=grid,
        in_specs=[
            pl.BlockSpec((_BR, D), lambda i: (i, 0)),
            pl.BlockSpec((D, H), lambda i: (0, 0)),
            pl.BlockSpec((1, H), lambda i: (0, 0)),
            pl.BlockSpec((H, OUT), lambda i: (0, 0)),
            pl.BlockSpec((1, OUT), lambda i: (0, 0)),
        ],
        out_specs=pl.BlockSpec(memory_space=pltpu.MemorySpace.HBM),
        out_shape=jax.ShapeDtypeStruct((NT, OUT), jnp.float32),
        scratch_shapes=[
            pltpu.VMEM((_Q, _BR, OUT), jnp.float32),
            pltpu.SemaphoreType.DMA((_Q,)),
        ],
    )(x_transaction, W_enc_t, b1, W_cls, b2)
